# baseline (speedup 1.0000x reference)
import functools

import jax
import jax.numpy as jnp
from jax import lax
from jax.experimental import pallas as pl
from jax.experimental.pallas import tpu as pltpu
from jax.experimental.pallas import tpu_sc as plsc

F = 26
V = 100000
D = 32
P = 128
B = 16384

G = 2
FG = F // G
GD = FG * D

NC = 2
NS = 16
NW = NC * NS
ROWS = B * FG
RPW = ROWS // NW
IPS = 128
NSTR = RPW // IPS
SPC = 2
CH = SPC * IPS
NCH = NSTR // SPC


def _sc_gather(tab_hbm, idx_hbm):
    mesh = plsc.VectorSubcoreMesh(core_axis_name="c", subcore_axis_name="s")

    @functools.partial(
        pl.kernel,
        mesh=mesh,
        out_type=jax.ShapeDtypeStruct((ROWS, D), jnp.float32),
        compiler_params=pltpu.CompilerParams(use_tc_tiling_on_sc=False),
        scratch_types=[
            pltpu.VMEM((NSTR, IPS), jnp.int32),
            pltpu.VMEM((CH, D), jnp.float32),
            pltpu.VMEM((CH, D), jnp.float32),
            pltpu.SemaphoreType.DMA,
            pltpu.SemaphoreType.DMA,
        ],
    )
    def k(tab, idx, out, idx_v, buf0, buf1, gsem0, gsem1):
        wid = lax.axis_index("s") * NC + lax.axis_index("c")
        pltpu.sync_copy(idx.at[wid], idx_v)
        row_base = wid * RPW

        bufs = (buf0, buf1)
        sems = (gsem0, gsem1)

        def fire(c, slot):
            for s in range(SPC):
                pltpu.async_copy(
                    tab.at[idx_v.at[c * SPC + s]],
                    bufs[slot].at[pl.ds(s * IPS, IPS)],
                    sems[slot],
                )

        def drain_and_store(c, slot):
            for s in range(SPC):
                pltpu.make_async_copy(
                    tab.at[idx_v.at[c * SPC + s]],
                    bufs[slot].at[pl.ds(s * IPS, IPS)],
                    sems[slot],
                ).wait()
            pltpu.sync_copy(bufs[slot], out.at[pl.ds(row_base + c * CH, CH)])

        fire(0, 0)

        def body(g, _):
            c = g * 2
            fire(c + 1, 1)
            drain_and_store(c, 0)

            @pl.when(c + 2 < NCH)
            def _():
                fire(c + 2, 0)

            drain_and_store(c + 1, 1)
            return 0

        lax.fori_loop(0, NCH // 2, body, 0)

    return k(tab_hbm, idx_hbm)


def _tc_body(x1_ref, x2_ref, s1_ref, s2_ref, e1_ref, e2_ref, g1_ref, g2_ref,
             bt1_ref, bt2_ref, w1a_ref, w1b_ref, b1_ref, w2_ref, b2_ref,
             o_ref):
    xs = [x1_ref[...], x2_ref[...]]
    ss = [s1_ref[...], s2_ref[...]]
    es = [e1_ref[...], e2_ref[...]]
    gs = [g1_ref[...], g2_ref[...]]
    bts = [bt1_ref[...], bt2_ref[...]]
    w1s = [w1a_ref[...], w1b_ref[...]]
    sum1 = None
    sum2 = None
    for x, s_m in zip(xs, ss):
        a = jnp.dot(x.astype(jnp.bfloat16), s_m,
                    preferred_element_type=jnp.float32)
        b = jnp.dot((x * x).astype(jnp.bfloat16), s_m,
                    preferred_element_type=jnp.float32)
        sum1 = a if sum1 is None else sum1 + a
        sum2 = b if sum2 is None else sum2 + b
    t = b1_ref[...].astype(jnp.float32)
    for x, e_m, g_m, bt_m, w1 in zip(xs, es, gs, bts, w1s):
        mu = jnp.dot(sum1.astype(jnp.bfloat16), e_m,
                     preferred_element_type=jnp.float32)
        m2 = jnp.dot(sum2.astype(jnp.bfloat16), e_m,
                     preferred_element_type=jnp.float32)
        var = m2 - mu * mu
        h = (x - mu) * lax.rsqrt(var + 1e-5)
        h = h * g_m + bt_m
        t = t + jnp.dot(h.astype(jnp.bfloat16), w1,
                        preferred_element_type=jnp.float32)
    u = 0.5 * t * (1.0 + lax.erf(t * 0.7071067811865476))
    o_ref[...] = jnp.dot(u.astype(jnp.bfloat16), w2_ref[...],
                         preferred_element_type=jnp.float32) + b2_ref[...]


def _tc_mlp(embs, s_ms, e_ms, gammas, betas, w1s, b1, w2, b2,
            interpret=False):
    BT = 256
    grid = (B // BT,)
    full = lambda shape: pl.BlockSpec(shape, lambda i: tuple(0 for _ in shape))
    in_specs = (
        [pl.BlockSpec((BT, GD), lambda i: (i, 0)) for _ in range(G)]
        + [full((GD, 128)) for _ in range(G)]
        + [full((128, GD)) for _ in range(G)]
        + [full((1, GD)) for _ in range(G)]
        + [full((1, GD)) for _ in range(G)]
        + [full((GD, P)) for _ in range(G)]
        + [full((1, P)), full((P, P)), full((1, P))]
    )
    return pl.pallas_call(
        _tc_body,
        grid=grid,
        in_specs=in_specs,
        out_specs=pl.BlockSpec((BT, P), lambda i: (i, 0)),
        out_shape=jax.ShapeDtypeStruct((B, P), jnp.float32),
        interpret=interpret,
    )(*embs, *s_ms, *e_ms, *gammas, *betas, *w1s, b1, w2, b2)


def _stats_mats(k):
    fid = jnp.arange(GD, dtype=jnp.int32) // D
    cols = jnp.arange(128, dtype=jnp.int32)
    s_m = jnp.where(fid[:, None] + k * FG == cols[None, :], 1.0 / D, 0.0)
    e_m = jnp.where(cols[:, None] == fid[None, :] + k * FG, 1.0, 0.0)
    return s_m.astype(jnp.bfloat16), e_m.astype(jnp.bfloat16)


def kernel(x_cat, tables, ln_gamma, ln_beta, W1, b1, W2, b2):
    embs, s_ms, e_ms, gammas, betas, w1s = [], [], [], [], [], []
    for k in range(G):
        fsl = slice(k * FG, (k + 1) * FG)
        flat = x_cat[:, fsl] + (jnp.arange(FG, dtype=jnp.int32) * V)[None, :]
        idx = flat.reshape(NW, NSTR, IPS)
        tab2 = tables[fsl].reshape(FG * V, D)
        emb = _sc_gather(tab2, idx).reshape(B, GD)
        embs.append(emb)
        s_m, e_m = _stats_mats(k)
        s_ms.append(s_m)
        e_ms.append(e_m)
        gammas.append(ln_gamma[fsl].reshape(1, GD))
        betas.append(ln_beta[fsl].reshape(1, GD))
        w1s.append(W1[k * GD:(k + 1) * GD].astype(jnp.bfloat16))
    return _tc_mlp(
        embs, s_ms, e_ms, gammas, betas, w1s,
        b1.reshape(1, P), W2.astype(jnp.bfloat16), b2.reshape(1, P),
    )


# speedup vs baseline: 1.5100x; 1.5100x over previous
import functools

import jax
import jax.numpy as jnp
from jax import lax
from jax.experimental import pallas as pl
from jax.experimental.pallas import tpu as pltpu
from jax.experimental.pallas import tpu_sc as plsc

F = 26
V = 100000
D = 32
P = 128
B = 16384

NC = 2
NS = 16
NW = NC * NS
ROWS = B * F
RPW = ROWS // NW
IPS = 128
NSTR = RPW // IPS
SPC = 4
CH = SPC * IPS
NCH = NSTR // SPC


def _sc_gather(tab_hbm, idx_hbm):
    mesh = plsc.VectorSubcoreMesh(core_axis_name="c", subcore_axis_name="s")

    @functools.partial(
        pl.kernel,
        mesh=mesh,
        out_type=jax.ShapeDtypeStruct((ROWS, D), jnp.float32),
        compiler_params=pltpu.CompilerParams(use_tc_tiling_on_sc=False),
        scratch_types=[
            pltpu.VMEM((NSTR, IPS), jnp.int32),
            pltpu.VMEM((CH, D), jnp.float32),
            pltpu.VMEM((CH, D), jnp.float32),
            pltpu.SemaphoreType.DMA,
            pltpu.SemaphoreType.DMA,
        ],
    )
    def k(tab, idx, out, idx_v, buf0, buf1, gsem0, gsem1):
        wid = lax.axis_index("s") * NC + lax.axis_index("c")
        pltpu.sync_copy(idx.at[wid], idx_v)
        row_base = wid * RPW

        bufs = (buf0, buf1)
        sems = (gsem0, gsem1)

        def fire(c, slot):
            for s in range(SPC):
                pltpu.async_copy(
                    tab.at[idx_v.at[c * SPC + s]],
                    bufs[slot].at[pl.ds(s * IPS, IPS)],
                    sems[slot],
                )

        def drain_and_store(c, slot):
            for s in range(SPC):
                pltpu.make_async_copy(
                    tab.at[idx_v.at[c * SPC + s]],
                    bufs[slot].at[pl.ds(s * IPS, IPS)],
                    sems[slot],
                ).wait()
            pltpu.sync_copy(bufs[slot], out.at[pl.ds(row_base + c * CH, CH)])

        fire(0, 0)

        def body(g, _):
            fire(g + 1, 1)
            drain_and_store(g, 0)

            @pl.when(g + 2 < NCH)
            def _():
                fire(g + 2, 0)

            drain_and_store(g + 1, 1)
            return 0

        lax.fori_loop(0, NCH // 2, lambda i, c: body(2 * i, c), 0)

    return k(tab_hbm, idx_hbm)


def _tc_body(x_ref, s_ref, e_ref, g_ref, bt_ref, w1_ref, b1_ref, w2_ref,
             b2_ref, o_ref):
    x = x_ref[...]
    xb = x.astype(jnp.bfloat16)
    s_m = s_ref[...]
    e_m = e_ref[...]
    s1 = jnp.dot(xb, s_m, preferred_element_type=jnp.float32)
    s2 = jnp.dot((x * x).astype(jnp.bfloat16), s_m,
                 preferred_element_type=jnp.float32)
    mu = jnp.dot(s1.astype(jnp.bfloat16), e_m,
                 preferred_element_type=jnp.float32)
    m2 = jnp.dot(s2.astype(jnp.bfloat16), e_m,
                 preferred_element_type=jnp.float32)
    var = m2 - mu * mu
    h = (x - mu) * lax.rsqrt(var + 1e-5)
    h = h * g_ref[...] + bt_ref[...]
    t = jnp.dot(h.astype(jnp.bfloat16), w1_ref[...],
                preferred_element_type=jnp.float32) + b1_ref[...]
    u = 0.5 * t * (1.0 + lax.erf(t * 0.7071067811865476))
    o_ref[...] = jnp.dot(u.astype(jnp.bfloat16), w2_ref[...],
                         preferred_element_type=jnp.float32) + b2_ref[...]


def _tc_mlp(emb, s_m, e_m, gamma, beta, w1, b1, w2, b2, interpret=False):
    BT = 256
    grid = (B // BT,)
    fd = F * D
    return pl.pallas_call(
        _tc_body,
        grid=grid,
        in_specs=[
            pl.BlockSpec((BT, fd), lambda i: (i, 0)),
            pl.BlockSpec((fd, 128), lambda i: (0, 0)),
            pl.BlockSpec((128, fd), lambda i: (0, 0)),
            pl.BlockSpec((1, fd), lambda i: (0, 0)),
            pl.BlockSpec((1, fd), lambda i: (0, 0)),
            pl.BlockSpec((fd, P), lambda i: (0, 0)),
            pl.BlockSpec((1, P), lambda i: (0, 0)),
            pl.BlockSpec((P, P), lambda i: (0, 0)),
            pl.BlockSpec((1, P), lambda i: (0, 0)),
        ],
        out_specs=pl.BlockSpec((BT, P), lambda i: (i, 0)),
        out_shape=jax.ShapeDtypeStruct((B, P), jnp.float32),
        interpret=interpret,
    )(emb, s_m, e_m, gamma, beta, w1, b1, w2, b2)


def _stats_mats():
    fd = F * D
    fid = jnp.arange(fd, dtype=jnp.int32) // D
    cols = jnp.arange(128, dtype=jnp.int32)
    s_m = jnp.where(fid[:, None] == cols[None, :], 1.0 / D, 0.0)
    e_m = jnp.where(cols[:, None] == fid[None, :], 1.0, 0.0)
    return s_m.astype(jnp.bfloat16), e_m.astype(jnp.bfloat16)


def kernel(x_cat, tables, ln_gamma, ln_beta, W1, b1, W2, b2):
    flat_idx = x_cat + (jnp.arange(F, dtype=jnp.int32) * V)[None, :]
    idx = flat_idx.reshape(NW, NSTR, IPS)
    tab2 = tables.reshape(F * V, D)
    emb = _sc_gather(tab2, idx).reshape(B, F * D)
    s_m, e_m = _stats_mats()
    return _tc_mlp(
        emb, s_m, e_m,
        ln_gamma.reshape(1, F * D), ln_beta.reshape(1, F * D),
        W1.astype(jnp.bfloat16), b1.reshape(1, P),
        W2.astype(jnp.bfloat16), b2.reshape(1, P),
    )
